# baseline (speedup 1.0000x reference)
import functools
import jax
import jax.numpy as jnp
from jax import lax
from jax.experimental import pallas as pl
from jax.experimental.pallas import tpu as pltpu
from jax.experimental.pallas import tpu_sc as plsc

_NUM_EXPERTS = 8
_BLOCK_ROWS = 4096
_N_TOKENS = 32768
_SC_CORES = 2
_SC_SUBCORES = 16
_SC_WORKERS = _SC_CORES * _SC_SUBCORES
_TOK_PER_W = _N_TOKENS // 2 // _SC_WORKERS
_VALS_PER_W = _TOK_PER_W * _NUM_EXPERTS


def _logits_block(hs_ref, wt_ref, bias_ref, out_ref):
    x = hs_ref[...]
    wt = wt_ref[...]
    out_ref[...] = jax.lax.dot_general(
        x, wt, (((1,), (0,)), ((), ())),
        preferred_element_type=jnp.float32,
    ) + bias_ref[...]


def _tc_logits(hs, weight, bias):
    n, hidden = hs.shape
    e = weight.shape[0]
    return pl.pallas_call(
        _logits_block,
        grid=(n // _BLOCK_ROWS,),
        in_specs=[
            pl.BlockSpec((_BLOCK_ROWS, hidden), lambda i: (i, 0)),
            pl.BlockSpec((hidden, e), lambda i: (0, 0)),
            pl.BlockSpec((1, e), lambda i: (0, 0)),
        ],
        out_specs=pl.BlockSpec((_BLOCK_ROWS, e), lambda i: (i, 0)),
        out_shape=jax.ShapeDtypeStruct((n, e), jnp.float32),
        compiler_params=pltpu.CompilerParams(
            dimension_semantics=("arbitrary",),
        ),
    )(hs, weight.T, bias.reshape(1, e))


_sc_mesh = plsc.VectorSubcoreMesh(
    core_axis_name="c", subcore_axis_name="s",
    num_cores=_SC_CORES, num_subcores=_SC_SUBCORES)


@functools.partial(
    pl.kernel,
    out_type=[
        jax.ShapeDtypeStruct((_N_TOKENS // 2 * _NUM_EXPERTS,), jnp.float32),
        jax.ShapeDtypeStruct((_N_TOKENS,), jnp.int32),
    ],
    mesh=_sc_mesh,
    compiler_params=pltpu.CompilerParams(needs_layout_passes=False),
    scratch_types=[
        pltpu.VMEM((_VALS_PER_W,), jnp.float32),
        pltpu.VMEM((_VALS_PER_W,), jnp.float32),
        pltpu.VMEM((_TOK_PER_W * 2,), jnp.int32),
    ],
)
def _sc_topk(logits_hbm, scores_hbm, idx_hbm, in_v, sc_v, ix_v):
    wid = lax.axis_index("s") * _SC_CORES + lax.axis_index("c")
    base = wid * _VALS_PER_W
    pltpu.sync_copy(logits_hbm.at[pl.ds(base, _VALS_PER_W)], in_v)
    tok16 = lax.iota(jnp.int32, 16)

    def body(g, carry):
        addr = g * 128 + tok16 * _NUM_EXPERTS
        keys = []
        for e in range(_NUM_EXPERTS):
            l = plsc.load_gather(in_v, [addr + e])
            b = lax.bitcast_convert_type(l, jnp.int32)
            keys.append(
                lax.bitcast_convert_type((b & -8) | (7 - e), jnp.float32))
        m1 = keys[0]
        for e in range(1, _NUM_EXPERTS):
            m1 = jnp.maximum(m1, keys[e])
        neginf = jnp.full((16,), -jnp.inf, jnp.float32)
        m2 = jnp.where(keys[0] == m1, neginf, keys[0])
        for e in range(1, _NUM_EXPERTS):
            m2 = jnp.maximum(m2, jnp.where(keys[e] == m1, neginf, keys[e]))
        m1b = lax.bitcast_convert_type(m1, jnp.int32)
        m2b = lax.bitcast_convert_type(m2, jnp.int32)
        v1 = lax.bitcast_convert_type(m1b & -8, jnp.float32)
        v2 = lax.bitcast_convert_type(m2b & -8, jnp.float32)
        z = jnp.exp(v2 - v1)
        s1 = 1.0 / (1.0 + z)
        s2 = z * s1
        zero = jnp.zeros((16,), jnp.float32)
        for e in range(_NUM_EXPERTS):
            sc = jnp.where(keys[e] == m1, s1,
                           jnp.where(keys[e] == m2, s2, zero))
            plsc.store_scatter(sc_v, [addr + e], sc)
        iaddr = g * 32 + tok16 * 2
        plsc.store_scatter(ix_v, [iaddr], 7 - (m1b & 7))
        plsc.store_scatter(ix_v, [iaddr + 1], 7 - (m2b & 7))
        return carry

    lax.fori_loop(0, _TOK_PER_W // 16, body, 0)
    pltpu.sync_copy(sc_v, scores_hbm.at[pl.ds(base, _VALS_PER_W)])
    pltpu.sync_copy(ix_v, idx_hbm.at[pl.ds(wid * _TOK_PER_W * 2,
                                           _TOK_PER_W * 2)])


@jax.jit
def kernel(hidden_states, weight, bias):
    hidden = weight.shape[1]
    hs = hidden_states.reshape(-1, hidden)
    n = hs.shape[0]
    e = weight.shape[0]
    half = n // 2
    out = []
    for h in range(2):
        logits = _tc_logits(hs[h * half:(h + 1) * half], weight, bias)
        out.append(_sc_topk(logits.reshape(-1)))
    scores = jnp.concatenate([o[0].reshape(half, e) for o in out], axis=0)
    idx = jnp.concatenate([o[1].reshape(half, 2) for o in out], axis=0)
    return scores, idx


# speedup vs baseline: 2.9702x; 2.9702x over previous
import functools
import jax
import jax.numpy as jnp
from jax.experimental import pallas as pl
from jax.experimental.pallas import tpu as pltpu

_NUM_EXPERTS = 8
_BLOCK_ROWS = 4096


def _router_block(hs_ref, wt_ref, bias_ref, scores_ref, idx_ref):
    x = hs_ref[...]
    wt = wt_ref[...]
    logits = jax.lax.dot_general(
        x, wt, (((1,), (0,)), ((), ())),
        preferred_element_type=jnp.float32,
    )
    logits = logits + bias_ref[...]
    r, e = logits.shape

    lane = jax.lax.broadcasted_iota(jnp.int32, (r, e), 1)
    bits = jax.lax.bitcast_convert_type(logits, jnp.int32)
    key = jax.lax.bitcast_convert_type((bits & -8) | (7 - lane), jnp.float32)

    m1 = jnp.max(key, axis=1, keepdims=True)
    is1 = key == m1
    m2 = jnp.max(jnp.where(is1, -jnp.inf, key), axis=1, keepdims=True)
    is2 = key == m2

    m1b = jax.lax.bitcast_convert_type(m1, jnp.int32)
    m2b = jax.lax.bitcast_convert_type(m2, jnp.int32)
    v1 = jax.lax.bitcast_convert_type(m1b & -8, jnp.float32)
    v2 = jax.lax.bitcast_convert_type(m2b & -8, jnp.float32)

    z = jnp.exp(v2 - v1)
    s1 = 1.0 / (1.0 + z)
    s2 = z * s1

    scores_ref[...] = jnp.where(is1, s1, jnp.where(is2, s2, 0.0))
    idx_ref[...] = jnp.concatenate(
        [7 - (m1b & 7), 7 - (m2b & 7)], axis=1)


@jax.jit
def kernel(hidden_states, weight, bias):
    hidden = weight.shape[1]
    hs = hidden_states.reshape(-1, hidden)
    n = hs.shape[0]
    e = weight.shape[0]
    grid = (n // _BLOCK_ROWS,)

    scores, indices = pl.pallas_call(
        _router_block,
        grid=grid,
        in_specs=[
            pl.BlockSpec((_BLOCK_ROWS, hidden), lambda i: (i, 0)),
            pl.BlockSpec((hidden, e), lambda i: (0, 0)),
            pl.BlockSpec((1, e), lambda i: (0, 0)),
        ],
        out_specs=[
            pl.BlockSpec((_BLOCK_ROWS, e), lambda i: (i, 0)),
            pl.BlockSpec((_BLOCK_ROWS, 2), lambda i: (i, 0)),
        ],
        out_shape=[
            jax.ShapeDtypeStruct((n, e), jnp.float32),
            jax.ShapeDtypeStruct((n, 2), jnp.int32),
        ],
        compiler_params=pltpu.CompilerParams(
            dimension_semantics=("parallel",),
        ),
    )(hs, weight.T, bias.reshape(1, e))
    return scores, indices
